# baseline (speedup 1.0000x reference)
import functools

import numpy as np

import jax
import jax.numpy as jnp
from jax import lax
from jax.experimental import pallas as pl
from jax.experimental.pallas import tpu as pltpu
from jax.experimental.pallas import tpu_sc as plsc

N = 10000
E = 320000
F_IN = 128
DIM = 16
HEADS = 8
OUT = 40
G = 16

NC = 2
NS = 16
NW = NC * NS

N_PAD = 10240
DUMMY = N
CHUNK = 128
CHUNKS = 82
EPW = CHUNK * CHUNKS
E_PAD = EPW * NW
ROWS_PER_TILE = N_PAD // NS

HH = HEADS // 2
W1ROW = 80
W2ROW = 32

_mesh = plsc.VectorSubcoreMesh(core_axis_name="c", subcore_axis_name="s",
                               num_cores=NC, num_subcores=NS)
_sc_params = pltpu.CompilerParams(use_tc_tiling_on_sc=False)


def _stage1_body(x_ref, w1_ref, asrc_ref, adst_ref,
                 xpa_ref, xpb_ref, as_ref, ad_ref):
    xp = jnp.dot(x_ref[...], w1_ref[...], preferred_element_type=jnp.float32)
    xpa_ref[...] = xp[:, :64]
    xpb_ref[...] = xp[:, 64:]
    as_ref[...] = jnp.dot(xp, asrc_ref[...], preferred_element_type=jnp.float32)
    ad_ref[...] = jnp.dot(xp, adst_ref[...], preferred_element_type=jnp.float32)


def _stage1(x_pad, W1, A_src, A_dst):
    bn = 1024
    return pl.pallas_call(
        _stage1_body,
        grid=(N_PAD // bn,),
        in_specs=[
            pl.BlockSpec((bn, F_IN), lambda i: (i, 0)),
            pl.BlockSpec((F_IN, F_IN), lambda i: (0, 0)),
            pl.BlockSpec((F_IN, 16), lambda i: (0, 0)),
            pl.BlockSpec((F_IN, 16), lambda i: (0, 0)),
        ],
        out_specs=[
            pl.BlockSpec((bn, 64), lambda i: (i, 0)),
            pl.BlockSpec((bn, 64), lambda i: (i, 0)),
            pl.BlockSpec((bn, 16), lambda i: (i, 0)),
            pl.BlockSpec((bn, 16), lambda i: (i, 0)),
        ],
        out_shape=[
            jax.ShapeDtypeStruct((N_PAD, 64), jnp.float32),
            jax.ShapeDtypeStruct((N_PAD, 64), jnp.float32),
            jax.ShapeDtypeStruct((N_PAD, 16), jnp.float32),
            jax.ShapeDtypeStruct((N_PAD, 16), jnp.float32),
        ],
    )(x_pad, W1, A_src, A_dst)


def _zero_rows(buf, width, rows):
    def zrow(e, _):
        for q in range(width // 16):
            buf[e, 16 * q:16 * q + 16] = jnp.zeros((16,), jnp.float32)
        return _
    lax.fori_loop(0, rows, zrow, None)


@functools.partial(
    pl.kernel, mesh=_mesh,
    out_type=jax.ShapeDtypeStruct((NC, 2 * N_PAD, W1ROW), jnp.float32),
    scratch_types=[
        pltpu.VMEM((CHUNKS, CHUNK), jnp.int32),
        pltpu.VMEM((CHUNKS, CHUNK), jnp.int32),
        pltpu.VMEM((CHUNK, 64), jnp.float32),
        pltpu.VMEM((CHUNK, 64), jnp.float32),
        pltpu.VMEM((CHUNK, 16), jnp.float32),
        pltpu.VMEM((CHUNK, 16), jnp.float32),
        pltpu.VMEM((CHUNK, 16), jnp.float32),
        pltpu.VMEM((CHUNK, 16), jnp.float32),
        pltpu.VMEM((CHUNK, W1ROW), jnp.float32),
        pltpu.VMEM_SHARED((N_PAD, W1ROW), jnp.float32),
        pltpu.SemaphoreType.DMA,
        pltpu.SemaphoreType.DMA,
    ],
    compiler_params=_sc_params)
def _edge_pass1(xpa_hbm, xpb_hbm, as_hbm, ad_hbm, src_hbm, dst_hbm, out_hbm,
                src_i, dst_i, xp0, xp1, as0, as1, ad0, ad1, out_b, acc,
                sem0, sem1):
    xp_b = (xp0, xp1)
    as_b = (as0, as1)
    ad_b = (ad0, ad1)
    sems = (sem0, sem1)
    c_id = lax.axis_index("c")
    s_id = lax.axis_index("s")
    w_id = c_id * NS + s_id
    arow = s_id * ROWS_PER_TILE

    pltpu.sync_copy(src_hbm.at[w_id], src_i)
    pltpu.sync_copy(dst_hbm.at[w_id], dst_i)

    for p in range(2):
        xp_tab = (xpa_hbm, xpb_hbm)[p]

        _zero_rows(out_b, W1ROW, CHUNK)

        def zacc(i, _):
            pltpu.sync_copy(out_b, acc.at[pl.ds(arow + i * CHUNK, CHUNK)])
            return _
        lax.fori_loop(0, ROWS_PER_TILE // CHUNK, zacc, None)
        plsc.subcore_barrier()

        def issue(c, slot):
            pltpu.async_copy(xp_tab.at[src_i.at[c]], xp_b[slot], sems[slot])
            pltpu.async_copy(as_hbm.at[src_i.at[c]], as_b[slot], sems[slot])
            pltpu.async_copy(ad_hbm.at[dst_i.at[c]], ad_b[slot], sems[slot])

        def wait(c, slot):
            pltpu.make_async_copy(xp_tab.at[src_i.at[c]], xp_b[slot],
                                  sems[slot]).wait()
            pltpu.make_async_copy(as_hbm.at[src_i.at[c]], as_b[slot],
                                  sems[slot]).wait()
            pltpu.make_async_copy(ad_hbm.at[dst_i.at[c]], ad_b[slot],
                                  sems[slot]).wait()

        def run_chunk(c, slot):
            wait(c, slot)

            def edge(e, _):
                al = as_b[slot][e, :] + ad_b[slot][e, :]
                al = jnp.maximum(al, 0.2 * al)
                ea = jnp.exp(al)
                out_b[e, 64:80] = ea
                for h in range(HH):
                    out_b[e, 16 * h:16 * h + 16] = (
                        xp_b[slot][e, 16 * h:16 * h + 16] * ea[HH * p + h])
                return _
            lax.fori_loop(0, CHUNK, edge, None, unroll=8)
            pltpu.sync_copy(out_b, acc.at[dst_i.at[c]], add=True)

        issue(0, 0)

        def outer(g, _):
            issue(2 * g + 1, 1)
            run_chunk(2 * g, 0)

            @pl.when(g < CHUNKS // 2 - 1)
            def _():
                issue(2 * g + 2, 0)

            run_chunk(2 * g + 1, 1)
            return _

        lax.fori_loop(0, CHUNKS // 2, outer, None)
        plsc.subcore_barrier()
        pltpu.sync_copy(acc.at[pl.ds(arow, ROWS_PER_TILE)],
                        out_hbm.at[c_id, pl.ds(p * N_PAD + arow,
                                               ROWS_PER_TILE)])


@functools.partial(
    pl.kernel, mesh=_mesh,
    out_type=jax.ShapeDtypeStruct((NC, N_PAD, W2ROW), jnp.float32),
    scratch_types=[
        pltpu.VMEM((CHUNKS, CHUNK), jnp.int32),
        pltpu.VMEM((CHUNKS, CHUNK), jnp.int32),
        pltpu.VMEM((CHUNK, 32), jnp.float32),
        pltpu.VMEM((CHUNK, 32), jnp.float32),
        pltpu.VMEM((CHUNK, 16), jnp.float32),
        pltpu.VMEM((CHUNK, 16), jnp.float32),
        pltpu.VMEM((CHUNK, W2ROW), jnp.float32),
        pltpu.VMEM_SHARED((N_PAD, W2ROW), jnp.float32),
        pltpu.SemaphoreType.DMA,
        pltpu.SemaphoreType.DMA,
    ],
    compiler_params=_sc_params)
def _edge_pass2(s2_hbm, d2_hbm, src_hbm, dst_hbm, out_hbm,
                src_i, dst_i, s20, s21, d20, d21, out_b, acc, sem0, sem1):
    s2_b = (s20, s21)
    d2_b = (d20, d21)
    sems = (sem0, sem1)
    c_id = lax.axis_index("c")
    s_id = lax.axis_index("s")
    w_id = c_id * NS + s_id
    arow = s_id * ROWS_PER_TILE

    pltpu.sync_copy(src_hbm.at[w_id], src_i)
    pltpu.sync_copy(dst_hbm.at[w_id], dst_i)

    _zero_rows(out_b, W2ROW, CHUNK)

    def zacc(i, _):
        pltpu.sync_copy(out_b, acc.at[pl.ds(arow + i * CHUNK, CHUNK)])
        return _
    lax.fori_loop(0, ROWS_PER_TILE // CHUNK, zacc, None)
    plsc.subcore_barrier()

    def issue(c, slot):
        pltpu.async_copy(s2_hbm.at[src_i.at[c]], s2_b[slot], sems[slot])
        pltpu.async_copy(d2_hbm.at[dst_i.at[c]], d2_b[slot], sems[slot])

    def wait(c, slot):
        pltpu.make_async_copy(s2_hbm.at[src_i.at[c]], s2_b[slot],
                              sems[slot]).wait()
        pltpu.make_async_copy(d2_hbm.at[dst_i.at[c]], d2_b[slot],
                              sems[slot]).wait()

    def run_chunk(c, slot):
        wait(c, slot)

        def edge(e, _):
            al = s2_b[slot][e, 16:32] + d2_b[slot][e, :]
            al = jnp.maximum(al, 0.2 * al)
            ea = jnp.exp(al)
            out_b[e, 16:32] = ea
            out_b[e, 0:16] = s2_b[slot][e, 0:16] * ea[0]
            return _
        lax.fori_loop(0, CHUNK, edge, None, unroll=8)
        pltpu.sync_copy(out_b, acc.at[dst_i.at[c]], add=True)

    issue(0, 0)

    def outer(g, _):
        issue(2 * g + 1, 1)
        run_chunk(2 * g, 0)

        @pl.when(g < CHUNKS // 2 - 1)
        def _():
            issue(2 * g + 2, 0)

        run_chunk(2 * g + 1, 1)
        return _

    lax.fori_loop(0, CHUNKS // 2, outer, None)
    plsc.subcore_barrier()
    pltpu.sync_copy(acc.at[pl.ds(arow, ROWS_PER_TILE)],
                    out_hbm.at[c_id, pl.ds(arow, ROWS_PER_TILE)])


def _stage2_body(acc_ref, b1_ref, bc8a_ref, bc8b_ref, w2a_ref, w2b_ref,
                 p_ref, q_ref, s2_ref, d2_ref):
    num_a = acc_ref[0, 0] + acc_ref[1, 0]
    num_b = acc_ref[0, 1] + acc_ref[1, 1]
    den = num_a[:, 64:72]
    r = 1.0 / (den + 1e-16)
    bca = jnp.dot(r, bc8a_ref[...], preferred_element_type=jnp.float32)
    bcb = jnp.dot(r, bc8b_ref[...], preferred_element_type=jnp.float32)
    ha = num_a[:, :64] * bca + b1_ref[:, :64]
    hb = num_b[:, :64] * bcb + b1_ref[:, 64:]
    ha = jnp.where(ha > 0, ha, jnp.exp(jnp.minimum(ha, 0.0)) - 1.0)
    hb = jnp.where(hb > 0, hb, jnp.exp(jnp.minimum(hb, 0.0)) - 1.0)
    xp2 = (jnp.dot(ha, w2a_ref[...], preferred_element_type=jnp.float32)
           + jnp.dot(hb, w2b_ref[...], preferred_element_type=jnp.float32))
    s2_ref[...] = jnp.dot(xp2, p_ref[...], preferred_element_type=jnp.float32)
    d2_ref[...] = jnp.dot(xp2, q_ref[...], preferred_element_type=jnp.float32)


def _stage2(acc1, b1, BC8, W2, P, Q):
    bn = 1024
    return pl.pallas_call(
        _stage2_body,
        grid=(N_PAD // bn,),
        in_specs=[
            pl.BlockSpec((NC, 2, bn, W1ROW), lambda i: (0, 0, i, 0)),
            pl.BlockSpec((1, F_IN), lambda i: (0, 0)),
            pl.BlockSpec((HEADS, 64), lambda i: (0, 0)),
            pl.BlockSpec((HEADS, 64), lambda i: (0, 0)),
            pl.BlockSpec((64, DIM), lambda i: (0, 0)),
            pl.BlockSpec((64, DIM), lambda i: (0, 0)),
            pl.BlockSpec((DIM, 32), lambda i: (0, 0)),
            pl.BlockSpec((DIM, 16), lambda i: (0, 0)),
        ],
        out_specs=[
            pl.BlockSpec((bn, 32), lambda i: (i, 0)),
            pl.BlockSpec((bn, 16), lambda i: (i, 0)),
        ],
        out_shape=[
            jax.ShapeDtypeStruct((N_PAD, 32), jnp.float32),
            jax.ShapeDtypeStruct((N_PAD, 16), jnp.float32),
        ],
    )(acc1, b1.reshape(1, F_IN), BC8[:, :64], BC8[:, 64:],
      W2[:64], W2[64:], P, Q)


def _stage3_body(acc_ref, batch_ref, b2_ref, lw1_ref, lb1_ref, lw2_ref,
                 lb2_ref, out_ref):
    num = acc_ref[0] + acc_ref[1]
    den = num[:, DIM:DIM + 1]
    h2 = num[:, :DIM] * (1.0 / (den + 1e-16)) + b2_ref[...]
    b = batch_ref[...]
    rows = []
    for g in range(G):
        rows.append(jnp.sum(jnp.where(b == g, h2, 0.0), axis=0, keepdims=True))
    pooled = jnp.concatenate(rows, axis=0)
    hid = jnp.maximum(
        jnp.dot(pooled, lw1_ref[...], preferred_element_type=jnp.float32)
        + lb1_ref[...], 0.0)
    out_ref[...] = (jnp.dot(hid, lw2_ref[...],
                            preferred_element_type=jnp.float32) + lb2_ref[...])


def _stage3(acc2, batch2d, b2, lw1, lb1, lw2, lb2):
    return pl.pallas_call(
        _stage3_body,
        out_shape=jax.ShapeDtypeStruct((G, OUT), jnp.float32),
    )(acc2, batch2d, b2.reshape(1, DIM), lw1, lb1.reshape(1, DIM),
      lw2, lb2.reshape(1, OUT))


def kernel(x, edge_index, batch, W1, a_src1, a_dst1, b1, W2, a_src2, a_dst2,
           b2, lw1, lb1, lw2, lb2):
    f32 = jnp.float32
    x_pad = jnp.pad(x.astype(f32), ((0, N_PAD - N), (0, 0)))

    hcol = np.repeat(np.arange(HEADS), DIM)
    m128x16 = (hcol[:, None] == np.arange(16)[None, :]).astype(np.float32)
    A_src = m128x16 * a_src1.reshape(-1)[:, None]
    A_dst = m128x16 * a_dst1.reshape(-1)[:, None]
    BC8 = jnp.asarray((np.arange(HEADS)[:, None] == hcol[None, :])
                      .astype(np.float32))
    eye16 = np.eye(DIM, dtype=np.float32)
    P = jnp.concatenate([jnp.asarray(eye16), a_src2.reshape(DIM, 1),
                         jnp.zeros((DIM, 32 - DIM - 1), f32)], axis=1)
    Q = jnp.concatenate([a_dst2.reshape(DIM, 1),
                         jnp.zeros((DIM, 15), f32)], axis=1)

    loop = jnp.arange(N, dtype=jnp.int32)
    pad_e = E_PAD - (E + N)
    src = jnp.concatenate([edge_index[0].astype(jnp.int32), loop,
                           jnp.zeros((pad_e,), jnp.int32)])
    dst = jnp.concatenate([edge_index[1].astype(jnp.int32), loop,
                           jnp.full((pad_e,), DUMMY, jnp.int32)])
    src3d = src.reshape(NW, CHUNKS, CHUNK)
    dst3d = dst.reshape(NW, CHUNKS, CHUNK)

    batch2d = jnp.pad(batch.astype(jnp.int32), (0, N_PAD - N),
                      constant_values=G).reshape(N_PAD, 1)

    xpa_t, xpb_t, as_t, ad_t = _stage1(x_pad, W1, A_src, A_dst)
    acc1 = _edge_pass1(xpa_t, xpb_t, as_t, ad_t, src3d, dst3d)
    acc1 = acc1.reshape(NC, 2, N_PAD, W1ROW)
    s2_t, d2_t = _stage2(acc1, b1, BC8, W2, P, Q)
    acc2 = _edge_pass2(s2_t, d2_t, src3d, dst3d)
    return _stage3(acc2, batch2d, b2, lw1, lb1, lw2, lb2)


# speedup vs baseline: 1.0915x; 1.0915x over previous
import functools

import numpy as np

import jax
import jax.numpy as jnp
from jax import lax
from jax.experimental import pallas as pl
from jax.experimental.pallas import tpu as pltpu
from jax.experimental.pallas import tpu_sc as plsc

N = 10000
E = 320000
F_IN = 128
DIM = 16
HEADS = 8
OUT = 40
G = 16

NC = 2
NS = 16
NW = NC * NS

N_PAD = 10240
DUMMY = N
CHUNK = 128
CHUNKS = 82
EPW = CHUNK * CHUNKS
E_PAD = EPW * NW
ROWS_PER_TILE = N_PAD // NS

HH = HEADS // 2
W1ROW = 80
W2ROW = 32

_mesh = plsc.VectorSubcoreMesh(core_axis_name="c", subcore_axis_name="s",
                               num_cores=NC, num_subcores=NS)
_sc_params = pltpu.CompilerParams(use_tc_tiling_on_sc=False)


def _stage1_body(x_ref, w1_ref, asrc_ref, adst_ref,
                 xpa_ref, xpb_ref, as_ref, ad_ref):
    xp = jnp.dot(x_ref[...], w1_ref[...], preferred_element_type=jnp.float32)
    xpa_ref[...] = xp[:, :64]
    xpb_ref[...] = xp[:, 64:]
    as_ref[...] = jnp.dot(xp, asrc_ref[...], preferred_element_type=jnp.float32)
    ad_ref[...] = jnp.dot(xp, adst_ref[...], preferred_element_type=jnp.float32)


def _stage1(x_pad, W1, A_src, A_dst):
    bn = 1024
    return pl.pallas_call(
        _stage1_body,
        grid=(N_PAD // bn,),
        in_specs=[
            pl.BlockSpec((bn, F_IN), lambda i: (i, 0)),
            pl.BlockSpec((F_IN, F_IN), lambda i: (0, 0)),
            pl.BlockSpec((F_IN, 16), lambda i: (0, 0)),
            pl.BlockSpec((F_IN, 16), lambda i: (0, 0)),
        ],
        out_specs=[
            pl.BlockSpec((bn, 64), lambda i: (i, 0)),
            pl.BlockSpec((bn, 64), lambda i: (i, 0)),
            pl.BlockSpec((bn, 16), lambda i: (i, 0)),
            pl.BlockSpec((bn, 16), lambda i: (i, 0)),
        ],
        out_shape=[
            jax.ShapeDtypeStruct((N_PAD, 64), jnp.float32),
            jax.ShapeDtypeStruct((N_PAD, 64), jnp.float32),
            jax.ShapeDtypeStruct((N_PAD, 16), jnp.float32),
            jax.ShapeDtypeStruct((N_PAD, 16), jnp.float32),
        ],
    )(x_pad, W1, A_src, A_dst)


def _zero_rows(buf, width, rows):
    def zrow(e, _):
        for q in range(width // 16):
            buf[e, 16 * q:16 * q + 16] = jnp.zeros((16,), jnp.float32)
        return _
    lax.fori_loop(0, rows, zrow, None)


@functools.partial(
    pl.kernel, mesh=_mesh,
    out_type=jax.ShapeDtypeStruct((NC, 2 * N_PAD, W1ROW), jnp.float32),
    scratch_types=[
        pltpu.VMEM((CHUNKS, CHUNK), jnp.int32),
        pltpu.VMEM((CHUNKS, CHUNK), jnp.int32),
        pltpu.VMEM((CHUNK, 64), jnp.float32),
        pltpu.VMEM((CHUNK, 64), jnp.float32),
        pltpu.VMEM((CHUNK, 16), jnp.float32),
        pltpu.VMEM((CHUNK, 16), jnp.float32),
        pltpu.VMEM((CHUNK, 16), jnp.float32),
        pltpu.VMEM((CHUNK, 16), jnp.float32),
        pltpu.VMEM((CHUNK, W1ROW), jnp.float32),
        pltpu.VMEM((CHUNK, W1ROW), jnp.float32),
        pltpu.VMEM_SHARED((N_PAD, W1ROW), jnp.float32),
        pltpu.SemaphoreType.DMA,
        pltpu.SemaphoreType.DMA,
        pltpu.SemaphoreType.DMA,
        pltpu.SemaphoreType.DMA,
    ],
    compiler_params=_sc_params)
def _edge_pass1(xpa_hbm, xpb_hbm, as_hbm, ad_hbm, src_hbm, dst_hbm, out_hbm,
                src_i, dst_i, xp0, xp1, as0, as1, ad0, ad1, ob0, ob1, acc,
                sem0, sem1, ssem0, ssem1):
    out_b = (ob0, ob1)
    ssems = (ssem0, ssem1)
    xp_b = (xp0, xp1)
    as_b = (as0, as1)
    ad_b = (ad0, ad1)
    sems = (sem0, sem1)
    c_id = lax.axis_index("c")
    s_id = lax.axis_index("s")
    w_id = c_id * NS + s_id
    arow = s_id * ROWS_PER_TILE

    pltpu.sync_copy(src_hbm.at[w_id], src_i)
    pltpu.sync_copy(dst_hbm.at[w_id], dst_i)

    for p in range(2):
        xp_tab = (xpa_hbm, xpb_hbm)[p]

        _zero_rows(out_b[0], W1ROW, CHUNK)

        def zacc(i, _):
            pltpu.sync_copy(out_b[0], acc.at[pl.ds(arow + i * CHUNK, CHUNK)])
            return _
        lax.fori_loop(0, ROWS_PER_TILE // CHUNK, zacc, None)
        plsc.subcore_barrier()

        def issue(c, slot):
            pltpu.async_copy(xp_tab.at[src_i.at[c]], xp_b[slot], sems[slot])
            pltpu.async_copy(as_hbm.at[src_i.at[c]], as_b[slot], sems[slot])
            pltpu.async_copy(ad_hbm.at[dst_i.at[c]], ad_b[slot], sems[slot])

        def wait(c, slot):
            pltpu.make_async_copy(xp_tab.at[src_i.at[c]], xp_b[slot],
                                  sems[slot]).wait()
            pltpu.make_async_copy(as_hbm.at[src_i.at[c]], as_b[slot],
                                  sems[slot]).wait()
            pltpu.make_async_copy(ad_hbm.at[dst_i.at[c]], ad_b[slot],
                                  sems[slot]).wait()

        def run_chunk(c, slot, scatter_pending):
            wait(c, slot)

            @pl.when(scatter_pending)
            def _():
                pltpu.make_async_copy(out_b[slot], acc.at[dst_i.at[c]],
                                      ssems[slot]).wait()

            def edge(e, _):
                al = as_b[slot][e, :] + ad_b[slot][e, :]
                al = jnp.maximum(al, 0.2 * al)
                ea = jnp.exp(al)
                out_b[slot][e, 64:80] = ea
                for h in range(HH):
                    out_b[slot][e, 16 * h:16 * h + 16] = (
                        xp_b[slot][e, 16 * h:16 * h + 16] * ea[HH * p + h])
                return _
            lax.fori_loop(0, CHUNK, edge, None)
            pltpu.async_copy(out_b[slot], acc.at[dst_i.at[c]], ssems[slot],
                             add=True)

        issue(0, 0)

        def outer(g, _):
            issue(2 * g + 1, 1)
            run_chunk(2 * g, 0, g >= 1)

            @pl.when(g < CHUNKS // 2 - 1)
            def _():
                issue(2 * g + 2, 0)

            run_chunk(2 * g + 1, 1, g >= 1)
            return _

        lax.fori_loop(0, CHUNKS // 2, outer, None)
        for slot in range(2):
            pltpu.make_async_copy(out_b[slot],
                                  acc.at[dst_i.at[CHUNKS - 2 + slot]],
                                  ssems[slot]).wait()
        plsc.subcore_barrier()
        pltpu.sync_copy(acc.at[pl.ds(arow, ROWS_PER_TILE)],
                        out_hbm.at[c_id, pl.ds(p * N_PAD + arow,
                                               ROWS_PER_TILE)])


@functools.partial(
    pl.kernel, mesh=_mesh,
    out_type=jax.ShapeDtypeStruct((NC, N_PAD, W2ROW), jnp.float32),
    scratch_types=[
        pltpu.VMEM((CHUNKS, CHUNK), jnp.int32),
        pltpu.VMEM((CHUNKS, CHUNK), jnp.int32),
        pltpu.VMEM((CHUNK, 32), jnp.float32),
        pltpu.VMEM((CHUNK, 32), jnp.float32),
        pltpu.VMEM((CHUNK, 16), jnp.float32),
        pltpu.VMEM((CHUNK, 16), jnp.float32),
        pltpu.VMEM((CHUNK, W2ROW), jnp.float32),
        pltpu.VMEM((CHUNK, W2ROW), jnp.float32),
        pltpu.VMEM_SHARED((N_PAD, W2ROW), jnp.float32),
        pltpu.SemaphoreType.DMA,
        pltpu.SemaphoreType.DMA,
        pltpu.SemaphoreType.DMA,
        pltpu.SemaphoreType.DMA,
    ],
    compiler_params=_sc_params)
def _edge_pass2(s2_hbm, d2_hbm, src_hbm, dst_hbm, out_hbm,
                src_i, dst_i, s20, s21, d20, d21, ob0, ob1, acc,
                sem0, sem1, ssem0, ssem1):
    out_b = (ob0, ob1)
    ssems = (ssem0, ssem1)
    s2_b = (s20, s21)
    d2_b = (d20, d21)
    sems = (sem0, sem1)
    c_id = lax.axis_index("c")
    s_id = lax.axis_index("s")
    w_id = c_id * NS + s_id
    arow = s_id * ROWS_PER_TILE

    pltpu.sync_copy(src_hbm.at[w_id], src_i)
    pltpu.sync_copy(dst_hbm.at[w_id], dst_i)

    _zero_rows(out_b[0], W2ROW, CHUNK)

    def zacc(i, _):
        pltpu.sync_copy(out_b[0], acc.at[pl.ds(arow + i * CHUNK, CHUNK)])
        return _
    lax.fori_loop(0, ROWS_PER_TILE // CHUNK, zacc, None)
    plsc.subcore_barrier()

    def issue(c, slot):
        pltpu.async_copy(s2_hbm.at[src_i.at[c]], s2_b[slot], sems[slot])
        pltpu.async_copy(d2_hbm.at[dst_i.at[c]], d2_b[slot], sems[slot])

    def wait(c, slot):
        pltpu.make_async_copy(s2_hbm.at[src_i.at[c]], s2_b[slot],
                              sems[slot]).wait()
        pltpu.make_async_copy(d2_hbm.at[dst_i.at[c]], d2_b[slot],
                              sems[slot]).wait()

    def run_chunk(c, slot, scatter_pending):
        wait(c, slot)

        @pl.when(scatter_pending)
        def _():
            pltpu.make_async_copy(out_b[slot], acc.at[dst_i.at[c]],
                                  ssems[slot]).wait()

        def edge(e, _):
            al = s2_b[slot][e, 16:32] + d2_b[slot][e, :]
            al = jnp.maximum(al, 0.2 * al)
            ea = jnp.exp(al)
            out_b[slot][e, 16:32] = ea
            out_b[slot][e, 0:16] = s2_b[slot][e, 0:16] * ea[0]
            return _
        lax.fori_loop(0, CHUNK, edge, None)
        pltpu.async_copy(out_b[slot], acc.at[dst_i.at[c]], ssems[slot],
                         add=True)

    issue(0, 0)

    def outer(g, _):
        issue(2 * g + 1, 1)
        run_chunk(2 * g, 0, g >= 1)

        @pl.when(g < CHUNKS // 2 - 1)
        def _():
            issue(2 * g + 2, 0)

        run_chunk(2 * g + 1, 1, g >= 1)
        return _

    lax.fori_loop(0, CHUNKS // 2, outer, None)
    for slot in range(2):
        pltpu.make_async_copy(out_b[slot],
                              acc.at[dst_i.at[CHUNKS - 2 + slot]],
                              ssems[slot]).wait()
    plsc.subcore_barrier()
    pltpu.sync_copy(acc.at[pl.ds(arow, ROWS_PER_TILE)],
                    out_hbm.at[c_id, pl.ds(arow, ROWS_PER_TILE)])


def _stage2_body(acc_ref, b1_ref, bc8a_ref, bc8b_ref, w2a_ref, w2b_ref,
                 p_ref, q_ref, s2_ref, d2_ref):
    num_a = acc_ref[0, 0] + acc_ref[1, 0]
    num_b = acc_ref[0, 1] + acc_ref[1, 1]
    den = num_a[:, 64:72]
    r = 1.0 / (den + 1e-16)
    bca = jnp.dot(r, bc8a_ref[...], preferred_element_type=jnp.float32)
    bcb = jnp.dot(r, bc8b_ref[...], preferred_element_type=jnp.float32)
    ha = num_a[:, :64] * bca + b1_ref[:, :64]
    hb = num_b[:, :64] * bcb + b1_ref[:, 64:]
    ha = jnp.where(ha > 0, ha, jnp.exp(jnp.minimum(ha, 0.0)) - 1.0)
    hb = jnp.where(hb > 0, hb, jnp.exp(jnp.minimum(hb, 0.0)) - 1.0)
    xp2 = (jnp.dot(ha, w2a_ref[...], preferred_element_type=jnp.float32)
           + jnp.dot(hb, w2b_ref[...], preferred_element_type=jnp.float32))
    s2_ref[...] = jnp.dot(xp2, p_ref[...], preferred_element_type=jnp.float32)
    d2_ref[...] = jnp.dot(xp2, q_ref[...], preferred_element_type=jnp.float32)


def _stage2(acc1, b1, BC8, W2, P, Q):
    bn = 1024
    return pl.pallas_call(
        _stage2_body,
        grid=(N_PAD // bn,),
        in_specs=[
            pl.BlockSpec((NC, 2, bn, W1ROW), lambda i: (0, 0, i, 0)),
            pl.BlockSpec((1, F_IN), lambda i: (0, 0)),
            pl.BlockSpec((HEADS, 64), lambda i: (0, 0)),
            pl.BlockSpec((HEADS, 64), lambda i: (0, 0)),
            pl.BlockSpec((64, DIM), lambda i: (0, 0)),
            pl.BlockSpec((64, DIM), lambda i: (0, 0)),
            pl.BlockSpec((DIM, 32), lambda i: (0, 0)),
            pl.BlockSpec((DIM, 16), lambda i: (0, 0)),
        ],
        out_specs=[
            pl.BlockSpec((bn, 32), lambda i: (i, 0)),
            pl.BlockSpec((bn, 16), lambda i: (i, 0)),
        ],
        out_shape=[
            jax.ShapeDtypeStruct((N_PAD, 32), jnp.float32),
            jax.ShapeDtypeStruct((N_PAD, 16), jnp.float32),
        ],
    )(acc1, b1.reshape(1, F_IN), BC8[:, :64], BC8[:, 64:],
      W2[:64], W2[64:], P, Q)


def _stage3_body(acc_ref, batch_ref, b2_ref, lw1_ref, lb1_ref, lw2_ref,
                 lb2_ref, out_ref):
    num = acc_ref[0] + acc_ref[1]
    den = num[:, DIM:DIM + 1]
    h2 = num[:, :DIM] * (1.0 / (den + 1e-16)) + b2_ref[...]
    b = batch_ref[...]
    rows = []
    for g in range(G):
        rows.append(jnp.sum(jnp.where(b == g, h2, 0.0), axis=0, keepdims=True))
    pooled = jnp.concatenate(rows, axis=0)
    hid = jnp.maximum(
        jnp.dot(pooled, lw1_ref[...], preferred_element_type=jnp.float32)
        + lb1_ref[...], 0.0)
    out_ref[...] = (jnp.dot(hid, lw2_ref[...],
                            preferred_element_type=jnp.float32) + lb2_ref[...])


def _stage3(acc2, batch2d, b2, lw1, lb1, lw2, lb2):
    return pl.pallas_call(
        _stage3_body,
        out_shape=jax.ShapeDtypeStruct((G, OUT), jnp.float32),
    )(acc2, batch2d, b2.reshape(1, DIM), lw1, lb1.reshape(1, DIM),
      lw2, lb2.reshape(1, OUT))


def kernel(x, edge_index, batch, W1, a_src1, a_dst1, b1, W2, a_src2, a_dst2,
           b2, lw1, lb1, lw2, lb2):
    f32 = jnp.float32
    x_pad = jnp.pad(x.astype(f32), ((0, N_PAD - N), (0, 0)))

    hcol = np.repeat(np.arange(HEADS), DIM)
    m128x16 = (hcol[:, None] == np.arange(16)[None, :]).astype(np.float32)
    A_src = m128x16 * a_src1.reshape(-1)[:, None]
    A_dst = m128x16 * a_dst1.reshape(-1)[:, None]
    BC8 = jnp.asarray((np.arange(HEADS)[:, None] == hcol[None, :])
                      .astype(np.float32))
    eye16 = np.eye(DIM, dtype=np.float32)
    P = jnp.concatenate([jnp.asarray(eye16), a_src2.reshape(DIM, 1),
                         jnp.zeros((DIM, 32 - DIM - 1), f32)], axis=1)
    Q = jnp.concatenate([a_dst2.reshape(DIM, 1),
                         jnp.zeros((DIM, 15), f32)], axis=1)

    loop = jnp.arange(N, dtype=jnp.int32)
    pad_e = E_PAD - (E + N)
    src = jnp.concatenate([edge_index[0].astype(jnp.int32), loop,
                           jnp.zeros((pad_e,), jnp.int32)])
    dst = jnp.concatenate([edge_index[1].astype(jnp.int32), loop,
                           jnp.full((pad_e,), DUMMY, jnp.int32)])
    src3d = src.reshape(NW, CHUNKS, CHUNK)
    dst3d = dst.reshape(NW, CHUNKS, CHUNK)

    batch2d = jnp.pad(batch.astype(jnp.int32), (0, N_PAD - N),
                      constant_values=G).reshape(N_PAD, 1)

    xpa_t, xpb_t, as_t, ad_t = _stage1(x_pad, W1, A_src, A_dst)
    acc1 = _edge_pass1(xpa_t, xpb_t, as_t, ad_t, src3d, dst3d)
    acc1 = acc1.reshape(NC, 2, N_PAD, W1ROW)
    s2_t, d2_t = _stage2(acc1, b1, BC8, W2, P, Q)
    acc2 = _edge_pass2(s2_t, d2_t, src3d, dst3d)
    return _stage3(acc2, batch2d, b2, lw1, lb1, lw2, lb2)


# speedup vs baseline: 1.6152x; 1.4798x over previous
import functools

import numpy as np

import jax
import jax.numpy as jnp
from jax import lax
from jax.experimental import pallas as pl
from jax.experimental.pallas import tpu as pltpu
from jax.experimental.pallas import tpu_sc as plsc

N = 10000
E = 320000
F_IN = 128
DIM = 16
HEADS = 8
OUT = 40
G = 16

NC = 2
NS = 16
NW = NC * NS

N_PAD = 10240
DUMMY = N
CHUNK = 128
CHUNKS = 82
EPW = CHUNK * CHUNKS
E_PAD = EPW * NW
ROWS_PER_TILE = N_PAD // NS

HH = HEADS // 2
W1ROW = 80
W2ROW = 32

_mesh = plsc.VectorSubcoreMesh(core_axis_name="c", subcore_axis_name="s",
                               num_cores=NC, num_subcores=NS)
_sc_params = pltpu.CompilerParams(use_tc_tiling_on_sc=False)


def _stage1_body(x_ref, w1_ref, asrc_ref, adst_ref,
                 xpa_ref, xpb_ref, as_ref, ad_ref):
    xp = jnp.dot(x_ref[...], w1_ref[...], preferred_element_type=jnp.float32)
    xpa_ref[...] = xp[:, :64]
    xpb_ref[...] = xp[:, 64:]
    as_ref[...] = jnp.dot(xp, asrc_ref[...], preferred_element_type=jnp.float32)
    ad_ref[...] = jnp.dot(xp, adst_ref[...], preferred_element_type=jnp.float32)


def _stage1(x_pad, W1, A_src, A_dst):
    bn = 1024
    return pl.pallas_call(
        _stage1_body,
        grid=(N_PAD // bn,),
        in_specs=[
            pl.BlockSpec((bn, F_IN), lambda i: (i, 0)),
            pl.BlockSpec((F_IN, F_IN), lambda i: (0, 0)),
            pl.BlockSpec((F_IN, 16), lambda i: (0, 0)),
            pl.BlockSpec((F_IN, 16), lambda i: (0, 0)),
        ],
        out_specs=[
            pl.BlockSpec((bn, 64), lambda i: (i, 0)),
            pl.BlockSpec((bn, 64), lambda i: (i, 0)),
            pl.BlockSpec((bn, 16), lambda i: (i, 0)),
            pl.BlockSpec((bn, 16), lambda i: (i, 0)),
        ],
        out_shape=[
            jax.ShapeDtypeStruct((N_PAD, 64), jnp.float32),
            jax.ShapeDtypeStruct((N_PAD, 64), jnp.float32),
            jax.ShapeDtypeStruct((N_PAD, 16), jnp.float32),
            jax.ShapeDtypeStruct((N_PAD, 16), jnp.float32),
        ],
    )(x_pad, W1, A_src, A_dst)


def _zero_rows(buf, width, rows):
    def zrow(e, _):
        for q in range(width // 16):
            buf[e, 16 * q:16 * q + 16] = jnp.zeros((16,), jnp.float32)
        return _
    lax.fori_loop(0, rows, zrow, None)


@functools.partial(
    pl.kernel, mesh=_mesh,
    out_type=jax.ShapeDtypeStruct((NC, 2 * N_PAD, W1ROW), jnp.float32),
    scratch_types=[
        pltpu.VMEM((CHUNKS, CHUNK), jnp.int32),
        pltpu.VMEM((CHUNKS, CHUNK), jnp.int32),
        pltpu.VMEM((CHUNK, 64), jnp.float32),
        pltpu.VMEM((CHUNK, 64), jnp.float32),
        pltpu.VMEM((CHUNK, 16), jnp.float32),
        pltpu.VMEM((CHUNK, 16), jnp.float32),
        pltpu.VMEM((CHUNK, 16), jnp.float32),
        pltpu.VMEM((CHUNK, 16), jnp.float32),
        pltpu.VMEM((CHUNK, W1ROW), jnp.float32),
        pltpu.VMEM((CHUNK, W1ROW), jnp.float32),
        pltpu.VMEM_SHARED((N_PAD, W1ROW), jnp.float32),
        pltpu.SemaphoreType.DMA,
        pltpu.SemaphoreType.DMA,
        pltpu.SemaphoreType.DMA,
        pltpu.SemaphoreType.DMA,
    ],
    compiler_params=_sc_params)
def _edge_pass1(xpa_hbm, xpb_hbm, as_hbm, ad_hbm, src_hbm, dst_hbm, out_hbm,
                src_i, dst_i, xp0, xp1, as0, as1, ad0, ad1, ob0, ob1, acc,
                sem0, sem1, ssem0, ssem1):
    out_b = (ob0, ob1)
    ssems = (ssem0, ssem1)
    xp_b = (xp0, xp1)
    as_b = (as0, as1)
    ad_b = (ad0, ad1)
    sems = (sem0, sem1)
    c_id = lax.axis_index("c")
    s_id = lax.axis_index("s")
    w_id = c_id * NS + s_id
    arow = s_id * ROWS_PER_TILE

    pltpu.sync_copy(src_hbm.at[w_id], src_i)
    pltpu.sync_copy(dst_hbm.at[w_id], dst_i)

    for p in range(2):
        xp_tab = (xpa_hbm, xpb_hbm)[p]

        _zero_rows(out_b[0], W1ROW, CHUNK)

        def zacc(i, _):
            pltpu.sync_copy(out_b[0], acc.at[pl.ds(arow + i * CHUNK, CHUNK)])
            return _
        lax.fori_loop(0, ROWS_PER_TILE // CHUNK, zacc, None)
        plsc.subcore_barrier()

        def issue(c, slot):
            pltpu.async_copy(xp_tab.at[src_i.at[c]], xp_b[slot], sems[slot])
            pltpu.async_copy(as_hbm.at[src_i.at[c]], as_b[slot], sems[slot])
            pltpu.async_copy(ad_hbm.at[dst_i.at[c]], ad_b[slot], sems[slot])

        def wait(c, slot):
            pltpu.make_async_copy(xp_tab.at[src_i.at[c]], xp_b[slot],
                                  sems[slot]).wait()
            pltpu.make_async_copy(as_hbm.at[src_i.at[c]], as_b[slot],
                                  sems[slot]).wait()
            pltpu.make_async_copy(ad_hbm.at[dst_i.at[c]], ad_b[slot],
                                  sems[slot]).wait()

        def run_chunk(c, slot, scatter_pending):
            wait(c, slot)

            @pl.when(scatter_pending)
            def _():
                pltpu.make_async_copy(out_b[slot], acc.at[dst_i.at[c]],
                                      ssems[slot]).wait()

            @plsc.parallel_loop(0, CHUNK, 1, unroll=4)
            def edge(e):
                al = as_b[slot][e, :] + ad_b[slot][e, :]
                al = jnp.maximum(al, 0.2 * al)
                ea = jnp.exp(al)
                out_b[slot][e, 64:80] = ea
                for h in range(HH):
                    out_b[slot][e, 16 * h:16 * h + 16] = (
                        xp_b[slot][e, 16 * h:16 * h + 16] * ea[HH * p + h])
            pltpu.async_copy(out_b[slot], acc.at[dst_i.at[c]], ssems[slot],
                             add=True)

        issue(0, 0)

        def outer(g, _):
            issue(2 * g + 1, 1)
            run_chunk(2 * g, 0, g >= 1)

            @pl.when(g < CHUNKS // 2 - 1)
            def _():
                issue(2 * g + 2, 0)

            run_chunk(2 * g + 1, 1, g >= 1)
            return _

        lax.fori_loop(0, CHUNKS // 2, outer, None)
        for slot in range(2):
            pltpu.make_async_copy(out_b[slot],
                                  acc.at[dst_i.at[CHUNKS - 2 + slot]],
                                  ssems[slot]).wait()
        plsc.subcore_barrier()
        pltpu.sync_copy(acc.at[pl.ds(arow, ROWS_PER_TILE)],
                        out_hbm.at[c_id, pl.ds(p * N_PAD + arow,
                                               ROWS_PER_TILE)])


@functools.partial(
    pl.kernel, mesh=_mesh,
    out_type=jax.ShapeDtypeStruct((NC, N_PAD, W2ROW), jnp.float32),
    scratch_types=[
        pltpu.VMEM((CHUNKS, CHUNK), jnp.int32),
        pltpu.VMEM((CHUNKS, CHUNK), jnp.int32),
        pltpu.VMEM((CHUNK, 32), jnp.float32),
        pltpu.VMEM((CHUNK, 32), jnp.float32),
        pltpu.VMEM((CHUNK, 16), jnp.float32),
        pltpu.VMEM((CHUNK, 16), jnp.float32),
        pltpu.VMEM((CHUNK, W2ROW), jnp.float32),
        pltpu.VMEM((CHUNK, W2ROW), jnp.float32),
        pltpu.VMEM_SHARED((N_PAD, W2ROW), jnp.float32),
        pltpu.SemaphoreType.DMA,
        pltpu.SemaphoreType.DMA,
        pltpu.SemaphoreType.DMA,
        pltpu.SemaphoreType.DMA,
    ],
    compiler_params=_sc_params)
def _edge_pass2(s2_hbm, d2_hbm, src_hbm, dst_hbm, out_hbm,
                src_i, dst_i, s20, s21, d20, d21, ob0, ob1, acc,
                sem0, sem1, ssem0, ssem1):
    out_b = (ob0, ob1)
    ssems = (ssem0, ssem1)
    s2_b = (s20, s21)
    d2_b = (d20, d21)
    sems = (sem0, sem1)
    c_id = lax.axis_index("c")
    s_id = lax.axis_index("s")
    w_id = c_id * NS + s_id
    arow = s_id * ROWS_PER_TILE

    pltpu.sync_copy(src_hbm.at[w_id], src_i)
    pltpu.sync_copy(dst_hbm.at[w_id], dst_i)

    _zero_rows(out_b[0], W2ROW, CHUNK)

    def zacc(i, _):
        pltpu.sync_copy(out_b[0], acc.at[pl.ds(arow + i * CHUNK, CHUNK)])
        return _
    lax.fori_loop(0, ROWS_PER_TILE // CHUNK, zacc, None)
    plsc.subcore_barrier()

    def issue(c, slot):
        pltpu.async_copy(s2_hbm.at[src_i.at[c]], s2_b[slot], sems[slot])
        pltpu.async_copy(d2_hbm.at[dst_i.at[c]], d2_b[slot], sems[slot])

    def wait(c, slot):
        pltpu.make_async_copy(s2_hbm.at[src_i.at[c]], s2_b[slot],
                              sems[slot]).wait()
        pltpu.make_async_copy(d2_hbm.at[dst_i.at[c]], d2_b[slot],
                              sems[slot]).wait()

    def run_chunk(c, slot, scatter_pending):
        wait(c, slot)

        @pl.when(scatter_pending)
        def _():
            pltpu.make_async_copy(out_b[slot], acc.at[dst_i.at[c]],
                                  ssems[slot]).wait()

        @plsc.parallel_loop(0, CHUNK, 1, unroll=4)
        def edge(e):
            al = s2_b[slot][e, 16:32] + d2_b[slot][e, :]
            al = jnp.maximum(al, 0.2 * al)
            ea = jnp.exp(al)
            out_b[slot][e, 16:32] = ea
            out_b[slot][e, 0:16] = s2_b[slot][e, 0:16] * ea[0]
        pltpu.async_copy(out_b[slot], acc.at[dst_i.at[c]], ssems[slot],
                         add=True)

    issue(0, 0)

    def outer(g, _):
        issue(2 * g + 1, 1)
        run_chunk(2 * g, 0, g >= 1)

        @pl.when(g < CHUNKS // 2 - 1)
        def _():
            issue(2 * g + 2, 0)

        run_chunk(2 * g + 1, 1, g >= 1)
        return _

    lax.fori_loop(0, CHUNKS // 2, outer, None)
    for slot in range(2):
        pltpu.make_async_copy(out_b[slot],
                              acc.at[dst_i.at[CHUNKS - 2 + slot]],
                              ssems[slot]).wait()
    plsc.subcore_barrier()
    pltpu.sync_copy(acc.at[pl.ds(arow, ROWS_PER_TILE)],
                    out_hbm.at[c_id, pl.ds(arow, ROWS_PER_TILE)])


def _stage2_body(acc_ref, b1_ref, bc8a_ref, bc8b_ref, w2a_ref, w2b_ref,
                 p_ref, q_ref, s2_ref, d2_ref):
    num_a = acc_ref[0, 0] + acc_ref[1, 0]
    num_b = acc_ref[0, 1] + acc_ref[1, 1]
    den = num_a[:, 64:72]
    r = 1.0 / (den + 1e-16)
    bca = jnp.dot(r, bc8a_ref[...], preferred_element_type=jnp.float32)
    bcb = jnp.dot(r, bc8b_ref[...], preferred_element_type=jnp.float32)
    ha = num_a[:, :64] * bca + b1_ref[:, :64]
    hb = num_b[:, :64] * bcb + b1_ref[:, 64:]
    ha = jnp.where(ha > 0, ha, jnp.exp(jnp.minimum(ha, 0.0)) - 1.0)
    hb = jnp.where(hb > 0, hb, jnp.exp(jnp.minimum(hb, 0.0)) - 1.0)
    xp2 = (jnp.dot(ha, w2a_ref[...], preferred_element_type=jnp.float32)
           + jnp.dot(hb, w2b_ref[...], preferred_element_type=jnp.float32))
    s2_ref[...] = jnp.dot(xp2, p_ref[...], preferred_element_type=jnp.float32)
    d2_ref[...] = jnp.dot(xp2, q_ref[...], preferred_element_type=jnp.float32)


def _stage2(acc1, b1, BC8, W2, P, Q):
    bn = 1024
    return pl.pallas_call(
        _stage2_body,
        grid=(N_PAD // bn,),
        in_specs=[
            pl.BlockSpec((NC, 2, bn, W1ROW), lambda i: (0, 0, i, 0)),
            pl.BlockSpec((1, F_IN), lambda i: (0, 0)),
            pl.BlockSpec((HEADS, 64), lambda i: (0, 0)),
            pl.BlockSpec((HEADS, 64), lambda i: (0, 0)),
            pl.BlockSpec((64, DIM), lambda i: (0, 0)),
            pl.BlockSpec((64, DIM), lambda i: (0, 0)),
            pl.BlockSpec((DIM, 32), lambda i: (0, 0)),
            pl.BlockSpec((DIM, 16), lambda i: (0, 0)),
        ],
        out_specs=[
            pl.BlockSpec((bn, 32), lambda i: (i, 0)),
            pl.BlockSpec((bn, 16), lambda i: (i, 0)),
        ],
        out_shape=[
            jax.ShapeDtypeStruct((N_PAD, 32), jnp.float32),
            jax.ShapeDtypeStruct((N_PAD, 16), jnp.float32),
        ],
    )(acc1, b1.reshape(1, F_IN), BC8[:, :64], BC8[:, 64:],
      W2[:64], W2[64:], P, Q)


def _stage3_body(acc_ref, batch_ref, b2_ref, lw1_ref, lb1_ref, lw2_ref,
                 lb2_ref, out_ref):
    num = acc_ref[0] + acc_ref[1]
    den = num[:, DIM:DIM + 1]
    h2 = num[:, :DIM] * (1.0 / (den + 1e-16)) + b2_ref[...]
    b = batch_ref[...]
    rows = []
    for g in range(G):
        rows.append(jnp.sum(jnp.where(b == g, h2, 0.0), axis=0, keepdims=True))
    pooled = jnp.concatenate(rows, axis=0)
    hid = jnp.maximum(
        jnp.dot(pooled, lw1_ref[...], preferred_element_type=jnp.float32)
        + lb1_ref[...], 0.0)
    out_ref[...] = (jnp.dot(hid, lw2_ref[...],
                            preferred_element_type=jnp.float32) + lb2_ref[...])


def _stage3(acc2, batch2d, b2, lw1, lb1, lw2, lb2):
    return pl.pallas_call(
        _stage3_body,
        out_shape=jax.ShapeDtypeStruct((G, OUT), jnp.float32),
    )(acc2, batch2d, b2.reshape(1, DIM), lw1, lb1.reshape(1, DIM),
      lw2, lb2.reshape(1, OUT))


def kernel(x, edge_index, batch, W1, a_src1, a_dst1, b1, W2, a_src2, a_dst2,
           b2, lw1, lb1, lw2, lb2):
    f32 = jnp.float32
    x_pad = jnp.pad(x.astype(f32), ((0, N_PAD - N), (0, 0)))

    hcol = np.repeat(np.arange(HEADS), DIM)
    m128x16 = (hcol[:, None] == np.arange(16)[None, :]).astype(np.float32)
    A_src = m128x16 * a_src1.reshape(-1)[:, None]
    A_dst = m128x16 * a_dst1.reshape(-1)[:, None]
    BC8 = jnp.asarray((np.arange(HEADS)[:, None] == hcol[None, :])
                      .astype(np.float32))
    eye16 = np.eye(DIM, dtype=np.float32)
    P = jnp.concatenate([jnp.asarray(eye16), a_src2.reshape(DIM, 1),
                         jnp.zeros((DIM, 32 - DIM - 1), f32)], axis=1)
    Q = jnp.concatenate([a_dst2.reshape(DIM, 1),
                         jnp.zeros((DIM, 15), f32)], axis=1)

    loop = jnp.arange(N, dtype=jnp.int32)
    pad_e = E_PAD - (E + N)
    src = jnp.concatenate([edge_index[0].astype(jnp.int32), loop,
                           jnp.zeros((pad_e,), jnp.int32)])
    dst = jnp.concatenate([edge_index[1].astype(jnp.int32), loop,
                           jnp.full((pad_e,), DUMMY, jnp.int32)])
    src3d = src.reshape(NW, CHUNKS, CHUNK)
    dst3d = dst.reshape(NW, CHUNKS, CHUNK)

    batch2d = jnp.pad(batch.astype(jnp.int32), (0, N_PAD - N),
                      constant_values=G).reshape(N_PAD, 1)

    xpa_t, xpb_t, as_t, ad_t = _stage1(x_pad, W1, A_src, A_dst)
    acc1 = _edge_pass1(xpa_t, xpb_t, as_t, ad_t, src3d, dst3d)
    acc1 = acc1.reshape(NC, 2, N_PAD, W1ROW)
    s2_t, d2_t = _stage2(acc1, b1, BC8, W2, P, Q)
    acc2 = _edge_pass2(s2_t, d2_t, src3d, dst3d)
    return _stage3(acc2, batch2d, b2, lw1, lb1, lw2, lb2)
